# baseline (speedup 1.0000x reference)
import functools

import jax
import jax.numpy as jnp
from jax import lax
from jax.experimental import pallas as pl
from jax.experimental.pallas import tpu as pltpu
from jax.experimental.pallas import tpu_sc as plsc

EMBED_DIM = 64
SCALE = 8.0


@functools.cache
def _build(B, V, D):
    info = plsc.get_sparse_core_info()
    NC, NS, L = info.num_cores, info.num_subcores, info.num_lanes
    NW = NC * NS
    assert B % NW == 0
    b_per_w = B // NW
    C = 128
    assert b_per_w % C == 0
    n_chunks = b_per_w // C

    mesh = plsc.VectorSubcoreMesh(core_axis_name="c", subcore_axis_name="s")

    @functools.partial(
        pl.kernel,
        mesh=mesh,
        out_type=jax.ShapeDtypeStruct((B, D), jnp.float32),
        scratch_types=[
            pltpu.VMEM((C,), jnp.int32),
            pltpu.VMEM((C, D), jnp.float32),
            pltpu.SemaphoreType.DMA,
        ],
        compiler_params=pltpu.CompilerParams(use_tc_tiling_on_sc=False),
    )
    def emb(idx_hbm, table_hbm, out_hbm, idx_v, rows_v, sem):
        wid = lax.axis_index("s") * NC + lax.axis_index("c")
        base = wid * b_per_w

        def chunk_body(i, carry):
            off = base + i * C
            pltpu.sync_copy(idx_hbm.at[pl.ds(off, C)], idx_v)
            pltpu.async_copy(table_hbm.at[idx_v], rows_v, sem).wait()

            def scale_body(r, c):
                for j in range(D // L):
                    sl = pl.ds(j * L, L)
                    rows_v[r, sl] = rows_v[r, sl] * SCALE
                return c

            lax.fori_loop(0, C, scale_body, carry)
            pltpu.sync_copy(rows_v, out_hbm.at[pl.ds(off, C)])
            return carry

        lax.fori_loop(0, n_chunks, chunk_body, 0)

    return emb


def kernel(inputs, table):
    B0, S = inputs.shape
    V, D = table.shape
    idx = inputs.reshape(B0 * S).astype(jnp.int32)
    out = _build(B0 * S, V, D)(idx, table)
    return out.reshape(B0, S, D)


# speedup vs baseline: 1.2793x; 1.2793x over previous
import functools

import jax
import jax.numpy as jnp
from jax import lax
from jax.experimental import pallas as pl
from jax.experimental.pallas import tpu as pltpu
from jax.experimental.pallas import tpu_sc as plsc

EMBED_DIM = 64
SCALE = 8.0
NBUF = 4
CHUNK = 128


@functools.cache
def _build(B, V, D):
    info = plsc.get_sparse_core_info()
    NC, NS, L = info.num_cores, info.num_subcores, info.num_lanes
    NW = NC * NS
    C = CHUNK
    assert B % (NW * C) == 0
    n_chunks = B // (NW * C)
    assert n_chunks % NBUF == 0
    n_grp = n_chunks // NBUF

    mesh = plsc.VectorSubcoreMesh(core_axis_name="c", subcore_axis_name="s")

    scratch = (
        [pltpu.VMEM((n_chunks, C), jnp.int32)]
        + [pltpu.VMEM((C, D), jnp.float32) for _ in range(2 * NBUF)]
        + [pltpu.SemaphoreType.DMA for _ in range(2 * NBUF)]
    )

    @functools.partial(
        pl.kernel,
        mesh=mesh,
        out_type=jax.ShapeDtypeStruct((B, D), jnp.float32),
        scratch_types=scratch,
        compiler_params=pltpu.CompilerParams(use_tc_tiling_on_sc=False),
    )
    def emb(idx_hbm, table_hbm, out_hbm, idx_v, *bufs):
        rows_g = list(bufs[:NBUF])
        rows_o = list(bufs[NBUF : 2 * NBUF])
        gsem = list(bufs[2 * NBUF : 3 * NBUF])
        osem = list(bufs[3 * NBUF : 4 * NBUF])

        wid = lax.axis_index("s") * NC + lax.axis_index("c")
        crow0 = wid * n_chunks

        pltpu.sync_copy(idx_hbm.at[pl.ds(crow0, n_chunks)], idx_v)

        def start_gather(b, i):
            pltpu.async_copy(table_hbm.at[idx_v.at[i]], rows_g[b], gsem[b])

        for b in range(NBUF):
            start_gather(b, b)

        def grp_body(gi, carry):
            for b in range(NBUF):
                i = gi * NBUF + b
                off = (crow0 + i) * C
                pltpu.make_async_copy(
                    table_hbm.at[idx_v.at[i]], rows_g[b], gsem[b]
                ).wait()
                @pl.when(gi >= 1)
                def _():
                    pltpu.make_async_copy(
                        rows_o[b], out_hbm.at[pl.ds(off, C)], osem[b]
                    ).wait()

                def scale_body(r4, c):
                    for rr in range(4):
                        r = r4 * 4 + rr
                        for j in range(D // L):
                            sl = pl.ds(j * L, L)
                            rows_o[b][r, sl] = rows_g[b][r, sl] * SCALE
                    return c

                lax.fori_loop(0, C // 4, scale_body, 0)
                pltpu.async_copy(rows_o[b], out_hbm.at[pl.ds(off, C)], osem[b])

                @pl.when(gi < n_grp - 1)
                def _():
                    start_gather(b, i + NBUF)

            return carry

        lax.fori_loop(0, n_grp, grp_body, 0)

        for b in range(NBUF):
            off = (crow0 + n_chunks - NBUF + b) * C
            pltpu.make_async_copy(
                rows_o[b], out_hbm.at[pl.ds(off, C)], osem[b]
            ).wait()

    return emb


def kernel(inputs, table):
    B0, S = inputs.shape
    V, D = table.shape
    B = B0 * S
    idx = inputs.reshape(B // CHUNK, CHUNK).astype(jnp.int32)
    out = _build(B, V, D)(idx, table)
    return out.reshape(B0, S, D)
